# baseline (speedup 1.0000x reference)
import jax
import jax.numpy as jnp
from jax import lax
from jax.experimental import pallas as pl
from jax.experimental.pallas import tpu as pltpu
from jax.experimental.pallas import tpu_sc as plsc

D_MODEL = 128
HIST = 50
HIST_PAD = 56
NBUF = 8
LAG = 6


def _gather_body(table_hbm, x_hbm, out_hbm, idx_v, rows_v, gsem, wsem):
    num_cores = 2
    wid = lax.axis_index("s") * num_cores + lax.axis_index("c")
    n_chunks = idx_v.shape[0]
    out_base = wid * n_chunks
    pltpu.sync_copy(x_hbm.at[pl.ds(wid * n_chunks, n_chunks)], idx_v)

    def start_gather(c, b):
        pltpu.async_copy(table_hbm.at[idx_v.at[c]], rows_v.at[b], gsem.at[b])

    def start_write(c, b):
        pltpu.async_copy(rows_v.at[b], out_hbm.at[out_base + c], wsem.at[b])

    def wait_gather(c, b):
        pltpu.make_async_copy(
            table_hbm.at[idx_v.at[c]], rows_v.at[b], gsem.at[b]).wait()

    def wait_write(b):
        pltpu.make_async_copy(
            rows_v.at[b], out_hbm.at[0], wsem.at[b]).wait()

    def step(c, b):
        if c_is_static := isinstance(c, int):
            assert b == c % NBUF
        if not c_is_static or c >= NBUF:
            wait_write(b)
        start_gather(c, b)
        d = (b - LAG) % NBUF
        if not c_is_static or c >= LAG:
            wait_gather(c - LAG, d)
            start_write(c - LAG, d)

    for c in range(NBUF):
        step(c, c % NBUF)

    n_main = (n_chunks - NBUF) // NBUF * NBUF
    def body(g, carry):
        c0 = NBUF + g * NBUF
        for j in range(NBUF):
            step(c0 + j, j)
        return carry
    lax.fori_loop(0, n_main // NBUF, body, 0)

    for c in range(NBUF + n_main, n_chunks):
        step(c, c % NBUF)

    for c in range(n_chunks - LAG, n_chunks):
        b = c % NBUF
        wait_gather(c, b)
        start_write(c, b)
    for b in range(NBUF):
        wait_write(b)


def kernel(x, table):
    batch, hist = x.shape
    info = plsc.get_sparse_core_info()
    nw = info.num_cores * info.num_subcores
    n_chunks = batch // nw

    mesh = plsc.VectorSubcoreMesh(core_axis_name="c", subcore_axis_name="s")
    run = pl.kernel(
        _gather_body,
        out_type=jax.ShapeDtypeStruct((batch, hist, D_MODEL), jnp.float32),
        mesh=mesh,
        scratch_types=[
            pltpu.VMEM((n_chunks, HIST), jnp.int32),
            pltpu.VMEM((NBUF, HIST, D_MODEL), jnp.float32),
            pltpu.SemaphoreType.DMA((NBUF,)),
            pltpu.SemaphoreType.DMA((NBUF,)),
        ],
    )
    return run(table, x)
